# baseline (speedup 1.0000x reference)
import jax
import jax.numpy as jnp
from jax import lax
from jax.experimental import pallas as pl
from jax.experimental.pallas import tpu as pltpu
from jax.experimental.pallas import tpu_sc as plsc

N = 10000
D = 128
E = 320000
NC = 2
NS = 16
CHUNK = 128
NCHUNKS = -(-E // (NS * CHUNK))
EPT = NCHUNKS * CHUNK
E_PAD = EPT * NS
N_PAD = 10240
HN = N_PAD // NC
TRASH = 128
ACC_ROWS = HN + TRASH
ART = ACC_ROWS // NS
OPT = HN // NS
RPT = N_PAD // NS
BLK = 128

_SC_MESH = plsc.VectorSubcoreMesh(
    core_axis_name="c", subcore_axis_name="s", num_cores=NC, num_subcores=NS)


def _deg_body(ei, part, deg_out, idx_v, deg_loc, tmp_v, sum_v):
    c = lax.axis_index("c")
    s = lax.axis_index("s")
    w = c * NS + s
    pltpu.sync_copy(ei.at[pl.ds(s * EPT, EPT)], idx_v.at[pl.ds(0, EPT)])
    pltpu.sync_copy(ei.at[pl.ds(E_PAD + s * EPT, EPT)],
                    idx_v.at[pl.ds(EPT, EPT)])

    one16 = jnp.ones((16,), jnp.float32)
    zero16 = jnp.zeros((16,), jnp.float32)

    def zloc(i, carry):
        deg_loc[pl.ds(i * 16, 16)] = zero16
        return carry

    lax.fori_loop(0, N_PAD // 16, zloc, 0)

    def hist(i, carry):
        v = idx_v[pl.ds(i * 16, 16)]
        plsc.addupdate_scatter(deg_loc, [v], one16)
        return carry

    lax.fori_loop(0, 2 * EPT // 16, hist, 0)

    pltpu.sync_copy(deg_loc, part.at[pl.ds(w * N_PAD, N_PAD)])
    plsc.subcore_barrier()

    def zsum(i, carry):
        sum_v[pl.ds(i * 16, 16)] = zero16
        return carry

    lax.fori_loop(0, RPT // 16, zsum, 0)

    for t in range(NS):
        pltpu.sync_copy(
            part.at[pl.ds((c * NS + t) * N_PAD + s * RPT, RPT)], tmp_v)

        def accum(i, carry):
            sum_v[pl.ds(i * 16, 16)] = (sum_v[pl.ds(i * 16, 16)]
                                        + tmp_v[pl.ds(i * 16, 16)])
            return carry

        lax.fori_loop(0, RPT // 16, accum, 0)

    pltpu.sync_copy(sum_v, deg_out.at[pl.ds(c * N_PAD + s * RPT, RPT)])


_sc_deg = pl.kernel(
    _deg_body,
    out_type=(
        jax.ShapeDtypeStruct((NC * NS * N_PAD,), jnp.float32),
        jax.ShapeDtypeStruct((NC * N_PAD,), jnp.float32),
    ),
    mesh=_SC_MESH,
    scratch_types=[
        pltpu.VMEM((2 * EPT,), jnp.int32),
        pltpu.VMEM((N_PAD,), jnp.float32),
        pltpu.VMEM((RPT,), jnp.float32),
        pltpu.VMEM((RPT,), jnp.float32),
    ],
    compiler_params=pltpu.CompilerParams(needs_layout_passes=False),
)


QE = EPT // 4
QSTEP = QE // 16
CAP = 5632
CAP_T = CAP + 512
NCH_T = CAP_T // CHUNK


def _spmm_body(ei, z, accp, qsrc, qdst, gidx, sflat, sidx2, rows_v, rows2_v,
               acc_sh, sem, sem2):
    c = lax.axis_index("c")
    s = lax.axis_index("s")

    lo16 = jnp.full((16,), HN, jnp.int32) * c.astype(jnp.int32)
    hn16 = jnp.full((16,), HN, jnp.int32)
    zero_i16 = jnp.zeros((16,), jnp.int32)
    zero16 = jnp.zeros((16,), jnp.float32)
    padg16 = jnp.full((16,), N, jnp.int32)
    pads16 = jnp.full((16,), HN, jnp.int32) + lax.iota(jnp.int32, 16)

    def fill(i, carry):
        for k in range(D // 16):
            rows_v[i, pl.ds(k * 16, 16)] = zero16
        return carry

    lax.fori_loop(0, CHUNK, fill, 0)

    for r in range(2):
        pltpu.sync_copy(rows_v,
                        acc_sh.at[pl.ds(s * ART + r * CHUNK, CHUNK)])
    pltpu.sync_copy(rows_v.at[pl.ds(0, ART - 2 * CHUNK)],
                    acc_sh.at[pl.ds(s * ART + 2 * CHUNK, ART - 2 * CHUNK)])
    plsc.subcore_barrier()

    for q in range(4):
        base = s * EPT + q * QE
        pltpu.sync_copy(ei.at[pl.ds(base, QE)], qsrc)
        pltpu.sync_copy(ei.at[pl.ds(E_PAD + base, QE)], qdst)

        def compact(gref, sref, off, carry):
            g = gref[pl.ds(off * 16, 16)]
            v = sref[pl.ds(off * 16, 16)]
            lv = v - lo16
            ok = (lv >= zero_i16) & (lv < hn16)
            oc = jnp.minimum(carry, CAP - 16)
            plsc.store_compressed(gidx.at[pl.ds(oc, 16)], g, mask=ok)
            plsc.store_compressed(sflat.at[pl.ds(oc, 16)], lv, mask=ok)
            return carry + jnp.sum(ok.astype(jnp.int32))

        off = lax.fori_loop(
            0, QSTEP, lambda i, o: compact(qsrc, qdst, i, o), 0)
        off = lax.fori_loop(
            0, QSTEP, lambda i, o: compact(qdst, qsrc, i, o), off)
        off = jnp.minimum(off, CAP)

        for k in range(16):
            gidx[pl.ds(off + k * 16, 16)] = padg16
            sflat[pl.ds(off + k * 16, 16)] = pads16

        nch = (off + CHUNK - 1) // CHUNK
        nch2 = nch + (nch % 2)

        def expand(j, carry):
            for k in range(CHUNK // 16):
                sidx2[j, pl.ds(k * 16, 16)] = sflat[pl.ds(j * CHUNK + k * 16,
                                                          16)]
            return carry

        lax.fori_loop(0, nch2, expand, 0)

        npair = nch2 // 2

        @pl.when(npair > 0)
        def _():
            pltpu.async_copy(z.at[gidx.at[pl.ds(0, CHUNK)]], rows_v, sem)

            def body(p, carry):
                pltpu.async_copy(
                    z.at[gidx.at[pl.ds((2 * p + 1) * CHUNK, CHUNK)]],
                    rows2_v, sem2)
                pltpu.make_async_copy(z.at[gidx.at[pl.ds(0, CHUNK)]],
                                      rows_v, sem).wait()
                pltpu.sync_copy(rows_v, acc_sh.at[sidx2.at[2 * p]], add=True)

                @pl.when(p < npair - 1)
                def _():
                    pltpu.async_copy(
                        z.at[gidx.at[pl.ds((2 * p + 2) * CHUNK, CHUNK)]],
                        rows_v, sem)

                pltpu.make_async_copy(z.at[gidx.at[pl.ds(0, CHUNK)]],
                                      rows2_v, sem2).wait()
                pltpu.sync_copy(rows2_v, acc_sh.at[sidx2.at[2 * p + 1]],
                                add=True)
                return carry

            lax.fori_loop(0, npair, body, 0)

    plsc.subcore_barrier()
    pltpu.sync_copy(acc_sh.at[pl.ds(s * OPT, OPT)],
                    accp.at[pl.ds(c * HN + s * OPT, OPT)])


_sc_spmm = pl.kernel(
    _spmm_body,
    out_type=jax.ShapeDtypeStruct((N_PAD, D), jnp.float32),
    mesh=_SC_MESH,
    scratch_types=[
        pltpu.VMEM((QE,), jnp.int32),
        pltpu.VMEM((QE,), jnp.int32),
        pltpu.VMEM((CAP_T,), jnp.int32),
        pltpu.VMEM((CAP_T,), jnp.int32),
        pltpu.VMEM((NCH_T, CHUNK), jnp.int32),
        pltpu.VMEM((CHUNK, D), jnp.float32),
        pltpu.VMEM((CHUNK, D), jnp.float32),
        pltpu.VMEM_SHARED((ACC_ROWS, D), jnp.float32),
        pltpu.SemaphoreType.DMA,
        pltpu.SemaphoreType.DMA,
    ],
    compiler_params=pltpu.CompilerParams(needs_layout_passes=False),
)


def _prep_body(x_ref, d0_ref, xn_ref, z0_ref, dinv_ref):
    x = x_ref[...]
    nrm = jnp.maximum(jnp.sqrt(jnp.sum(x * x, axis=1, keepdims=True)), 1e-12)
    xn = x / nrm
    deg = d0_ref[...] + 1.0
    dinv = lax.rsqrt(deg)
    xn_ref[...] = xn
    z0_ref[...] = xn * dinv
    dinv_ref[...] = dinv


_row = pl.BlockSpec((BLK, D), lambda i: (i, 0))
_col = pl.BlockSpec((BLK, 1), lambda i: (i, 0))
_wmat = pl.BlockSpec((D, D), lambda i: (0, 0))
_brow = pl.BlockSpec((1, D), lambda i: (0, 0))

_tc_prep = pl.pallas_call(
    _prep_body,
    grid=(N_PAD // BLK,),
    in_specs=[_row, _col],
    out_specs=[_row, _row, _col],
    out_shape=[
        jax.ShapeDtypeStruct((N_PAD, D), jnp.float32),
        jax.ShapeDtypeStruct((N_PAD, D), jnp.float32),
        jax.ShapeDtypeStruct((N_PAD, 1), jnp.float32),
    ],
)


def _layer_body(acc, z, dinv, s_ref, wa, ba, wb, bb, zn_ref, sn_ref):
    dv = dinv[...]
    y = (acc[...] + z[...]) * dv
    ha = jnp.dot(y, wa[...], preferred_element_type=jnp.float32) + ba[...]
    xa = jnp.where(ha >= 0, ha, 0.01 * ha)
    xb = jnp.dot(y, wb[...], preferred_element_type=jnp.float32) + bb[...]
    zn_ref[...] = xa * dv
    sn_ref[...] = s_ref[...] + xa + xb


_tc_layer = pl.pallas_call(
    _layer_body,
    grid=(N_PAD // BLK,),
    in_specs=[_row, _row, _col, _row, _wmat, _brow, _wmat, _brow],
    out_specs=[_row, _row],
    out_shape=[
        jax.ShapeDtypeStruct((N_PAD, D), jnp.float32),
        jax.ShapeDtypeStruct((N_PAD, D), jnp.float32),
    ],
)


def kernel(edge_index, id_embedding, W1, b1, W2, b2, W3, b3):
    pad = jnp.full((2, E_PAD - E), N, dtype=jnp.int32)
    ei = jnp.concatenate([edge_index, pad], axis=1).reshape(2 * E_PAD)
    x_pad = jnp.pad(id_embedding, ((0, N_PAD - N), (0, 0)))

    _, deg_flat = _sc_deg(ei)
    d0 = deg_flat[:N_PAD].reshape(N_PAD, 1)
    xn, z0, dinv = _tc_prep(x_pad, d0)

    was = jnp.stack([W1, W2])
    bas = jnp.stack([b1.reshape(1, D), b2.reshape(1, D)])
    wbs = jnp.stack([jnp.zeros_like(W3), W3])
    bbs = jnp.stack([jnp.zeros((1, D), jnp.float32), b3.reshape(1, D)])

    def step(carry, xs):
        z, s = carry
        wa, ba, wb, bb = xs
        acc = _sc_spmm(ei, z)
        zn, sn = _tc_layer(acc, z, dinv, s, wa, ba, wb, bb)
        return (zn, sn), None

    c1, _ = step((z0, xn), (was[0], bas[0], wbs[0], bbs[0]))
    (_, s_fin), _ = step(c1, (was[1], bas[1], wbs[1], bbs[1]))
    return s_fin[:N]


# speedup vs baseline: 1.7302x; 1.7302x over previous
import jax
import jax.numpy as jnp
from jax import lax
from jax.experimental import pallas as pl
from jax.experimental.pallas import tpu as pltpu
from jax.experimental.pallas import tpu_sc as plsc

N = 10000
D = 128
E = 320000
NC = 2
NS = 16
CHUNK = 128
NCHUNKS = -(-E // (NS * CHUNK))
EPT = NCHUNKS * CHUNK
E_PAD = EPT * NS
N_PAD = 10240
HN = N_PAD // NC
TRASH = 128
ACC_ROWS = HN + TRASH
ART = ACC_ROWS // NS
OPT = HN // NS
RPT = N_PAD // NS
BLK = 128

_SC_MESH = plsc.VectorSubcoreMesh(
    core_axis_name="c", subcore_axis_name="s", num_cores=NC, num_subcores=NS)


def _deg_body(ei, part, deg_out, idx_v, deg_loc, tmp_v, sum_v):
    c = lax.axis_index("c")
    s = lax.axis_index("s")
    w = c * NS + s
    pltpu.sync_copy(ei.at[pl.ds(s * EPT, EPT)], idx_v.at[pl.ds(0, EPT)])
    pltpu.sync_copy(ei.at[pl.ds(E_PAD + s * EPT, EPT)],
                    idx_v.at[pl.ds(EPT, EPT)])

    one16 = jnp.ones((16,), jnp.float32)
    zero16 = jnp.zeros((16,), jnp.float32)

    def zloc(i, carry):
        deg_loc[pl.ds(i * 16, 16)] = zero16
        return carry

    lax.fori_loop(0, N_PAD // 16, zloc, 0)

    def hist(i, carry):
        v = idx_v[pl.ds(i * 16, 16)]
        plsc.addupdate_scatter(deg_loc, [v], one16)
        return carry

    lax.fori_loop(0, 2 * EPT // 16, hist, 0)

    pltpu.sync_copy(deg_loc, part.at[pl.ds(w * N_PAD, N_PAD)])
    plsc.subcore_barrier()

    def zsum(i, carry):
        sum_v[pl.ds(i * 16, 16)] = zero16
        return carry

    lax.fori_loop(0, RPT // 16, zsum, 0)

    for t in range(NS):
        pltpu.sync_copy(
            part.at[pl.ds((c * NS + t) * N_PAD + s * RPT, RPT)], tmp_v)

        def accum(i, carry):
            sum_v[pl.ds(i * 16, 16)] = (sum_v[pl.ds(i * 16, 16)]
                                        + tmp_v[pl.ds(i * 16, 16)])
            return carry

        lax.fori_loop(0, RPT // 16, accum, 0)

    pltpu.sync_copy(sum_v, deg_out.at[pl.ds(c * N_PAD + s * RPT, RPT)])


_sc_deg = pl.kernel(
    _deg_body,
    out_type=(
        jax.ShapeDtypeStruct((NC * NS * N_PAD,), jnp.float32),
        jax.ShapeDtypeStruct((NC * N_PAD,), jnp.float32),
    ),
    mesh=_SC_MESH,
    scratch_types=[
        pltpu.VMEM((2 * EPT,), jnp.int32),
        pltpu.VMEM((N_PAD,), jnp.float32),
        pltpu.VMEM((RPT,), jnp.float32),
        pltpu.VMEM((RPT,), jnp.float32),
    ],
    compiler_params=pltpu.CompilerParams(needs_layout_passes=False),
)


def _spmm_body(ei, z, accp, idx_v, idx2_v, rows_v, rows2_v, acc_sh, sem,
               sem2, sem_s, sem_s2):
    c = lax.axis_index("c")
    s = lax.axis_index("s")
    pltpu.sync_copy(ei.at[pl.ds(s * EPT, EPT)], idx_v.at[pl.ds(0, EPT)])
    pltpu.sync_copy(ei.at[pl.ds(E_PAD + s * EPT, EPT)],
                    idx_v.at[pl.ds(EPT, EPT)])

    lo16 = jnp.full((16,), HN, jnp.int32) * c.astype(jnp.int32)
    hn16 = jnp.full((16,), HN, jnp.int32)
    msk16 = jnp.full((16,), TRASH - 1, jnp.int32)
    zero_i16 = jnp.zeros((16,), jnp.int32)
    zero16 = jnp.zeros((16,), jnp.float32)

    def fill(i, carry):
        for k in range(D // 16):
            rows_v[i, pl.ds(k * 16, 16)] = zero16
        return carry

    lax.fori_loop(0, CHUNK, fill, 0)

    for r in range(2):
        pltpu.sync_copy(rows_v,
                        acc_sh.at[pl.ds(s * ART + r * CHUNK, CHUNK)])
    pltpu.sync_copy(rows_v.at[pl.ds(0, ART - 2 * CHUNK)],
                    acc_sh.at[pl.ds(s * ART + 2 * CHUNK, ART - 2 * CHUNK)])
    plsc.subcore_barrier()

    def localize(base, row):
        for k in range(CHUNK // 16):
            v = idx_v[pl.ds(base + k * 16, 16)]
            lv = v - lo16
            ok = (lv >= zero_i16) & (lv < hn16)
            tr = hn16 + (v & msk16)
            idx2_v[row, pl.ds(k * 16, 16)] = jnp.where(ok, lv, tr)

    pltpu.async_copy(z.at[idx_v.at[pl.ds(0, CHUNK)]], rows_v, sem)

    def body(j, carry):
        pltpu.async_copy(z.at[idx_v.at[pl.ds(EPT + j * CHUNK, CHUNK)]],
                         rows2_v, sem2)
        localize(EPT + j * CHUNK, 0)
        pltpu.make_async_copy(z.at[idx_v.at[pl.ds(0, CHUNK)]], rows_v,
                              sem).wait()
        pltpu.sync_copy(rows_v, acc_sh.at[idx2_v.at[0]], add=True)

        @pl.when(j < NCHUNKS - 1)
        def _():
            pltpu.async_copy(
                z.at[idx_v.at[pl.ds((j + 1) * CHUNK, CHUNK)]], rows_v, sem)

        localize(j * CHUNK, 1)
        pltpu.make_async_copy(z.at[idx_v.at[pl.ds(0, CHUNK)]], rows2_v,
                              sem2).wait()
        pltpu.sync_copy(rows2_v, acc_sh.at[idx2_v.at[1]], add=True)
        return carry

    lax.fori_loop(0, NCHUNKS, body, 0)
    plsc.subcore_barrier()
    pltpu.sync_copy(acc_sh.at[pl.ds(s * OPT, OPT)],
                    accp.at[pl.ds(c * HN + s * OPT, OPT)])


_sc_spmm = pl.kernel(
    _spmm_body,
    out_type=jax.ShapeDtypeStruct((N_PAD, D), jnp.float32),
    mesh=_SC_MESH,
    scratch_types=[
        pltpu.VMEM((2 * EPT,), jnp.int32),
        pltpu.VMEM((2, CHUNK), jnp.int32),
        pltpu.VMEM((CHUNK, D), jnp.float32),
        pltpu.VMEM((CHUNK, D), jnp.float32),
        pltpu.VMEM_SHARED((ACC_ROWS, D), jnp.float32),
        pltpu.SemaphoreType.DMA,
        pltpu.SemaphoreType.DMA,
        pltpu.SemaphoreType.DMA,
        pltpu.SemaphoreType.DMA,
    ],
)


def _prep_body(x_ref, d0_ref, xn_ref, z0_ref, dinv_ref):
    x = x_ref[...]
    nrm = jnp.maximum(jnp.sqrt(jnp.sum(x * x, axis=1, keepdims=True)), 1e-12)
    xn = x / nrm
    deg = d0_ref[...] + 1.0
    dinv = lax.rsqrt(deg)
    xn_ref[...] = xn
    z0_ref[...] = xn * dinv
    dinv_ref[...] = dinv


_row = pl.BlockSpec((BLK, D), lambda i: (i, 0))
_col = pl.BlockSpec((BLK, 1), lambda i: (i, 0))
_wmat = pl.BlockSpec((D, D), lambda i: (0, 0))
_brow = pl.BlockSpec((1, D), lambda i: (0, 0))

_tc_prep = pl.pallas_call(
    _prep_body,
    grid=(N_PAD // BLK,),
    in_specs=[_row, _col],
    out_specs=[_row, _row, _col],
    out_shape=[
        jax.ShapeDtypeStruct((N_PAD, D), jnp.float32),
        jax.ShapeDtypeStruct((N_PAD, D), jnp.float32),
        jax.ShapeDtypeStruct((N_PAD, 1), jnp.float32),
    ],
)


def _layer_body(acc, z, dinv, s_ref, wa, ba, wb, bb, zn_ref, sn_ref):
    dv = dinv[...]
    y = (acc[...] + z[...]) * dv
    ha = jnp.dot(y, wa[...], preferred_element_type=jnp.float32) + ba[...]
    xa = jnp.where(ha >= 0, ha, 0.01 * ha)
    xb = jnp.dot(y, wb[...], preferred_element_type=jnp.float32) + bb[...]
    zn_ref[...] = xa * dv
    sn_ref[...] = s_ref[...] + xa + xb


_tc_layer = pl.pallas_call(
    _layer_body,
    grid=(N_PAD // BLK,),
    in_specs=[_row, _row, _col, _row, _wmat, _brow, _wmat, _brow],
    out_specs=[_row, _row],
    out_shape=[
        jax.ShapeDtypeStruct((N_PAD, D), jnp.float32),
        jax.ShapeDtypeStruct((N_PAD, D), jnp.float32),
    ],
)


def kernel(edge_index, id_embedding, W1, b1, W2, b2, W3, b3):
    pad = jnp.full((2, E_PAD - E), N, dtype=jnp.int32)
    ei = jnp.concatenate([edge_index, pad], axis=1).reshape(2 * E_PAD)
    x_pad = jnp.pad(id_embedding, ((0, N_PAD - N), (0, 0)))

    _, deg_flat = _sc_deg(ei)
    d0 = deg_flat[:N_PAD].reshape(N_PAD, 1)
    xn, z0, dinv = _tc_prep(x_pad, d0)

    was = jnp.stack([W1, W2])
    bas = jnp.stack([b1.reshape(1, D), b2.reshape(1, D)])
    wbs = jnp.stack([jnp.zeros_like(W3), W3])
    bbs = jnp.stack([jnp.zeros((1, D), jnp.float32), b3.reshape(1, D)])

    def step(carry, xs):
        z, s = carry
        wa, ba, wb, bb = xs
        acc = _sc_spmm(ei, z)
        zn, sn = _tc_layer(acc, z, dinv, s, wa, ba, wb, bb)
        return (zn, sn), None

    c1, _ = step((z0, xn), (was[0], bas[0], wbs[0], bbs[0]))
    (_, s_fin), _ = step(c1, (was[1], bas[1], wbs[1], bbs[1]))
    return s_fin[:N]


# speedup vs baseline: 1.7304x; 1.0001x over previous
import jax
import jax.numpy as jnp
from jax import lax
from jax.experimental import pallas as pl
from jax.experimental.pallas import tpu as pltpu
from jax.experimental.pallas import tpu_sc as plsc

N = 10000
D = 128
E = 320000
NC = 2
NS = 16
CHUNK = 128
NCHUNKS = -(-E // (NS * CHUNK))
EPT = NCHUNKS * CHUNK
E_PAD = EPT * NS
N_PAD = 10240
HN = N_PAD // NC
TRASH = 128
ACC_ROWS = HN + TRASH
ART = ACC_ROWS // NS
OPT = HN // NS
RPT = N_PAD // NS
BLK = 128

_SC_MESH = plsc.VectorSubcoreMesh(
    core_axis_name="c", subcore_axis_name="s", num_cores=NC, num_subcores=NS)


def _deg_body(ei, part, deg_out, idx_v, deg_loc, tmp_v, sum_v):
    c = lax.axis_index("c")
    s = lax.axis_index("s")
    w = c * NS + s
    pltpu.sync_copy(ei.at[pl.ds(s * EPT, EPT)], idx_v.at[pl.ds(0, EPT)])
    pltpu.sync_copy(ei.at[pl.ds(E_PAD + s * EPT, EPT)],
                    idx_v.at[pl.ds(EPT, EPT)])

    one16 = jnp.ones((16,), jnp.float32)
    zero16 = jnp.zeros((16,), jnp.float32)

    def zloc(i, carry):
        deg_loc[pl.ds(i * 16, 16)] = zero16
        return carry

    lax.fori_loop(0, N_PAD // 16, zloc, 0)

    def hist(i, carry):
        v = idx_v[pl.ds(i * 16, 16)]
        plsc.addupdate_scatter(deg_loc, [v], one16)
        return carry

    lax.fori_loop(0, 2 * EPT // 16, hist, 0)

    pltpu.sync_copy(deg_loc, part.at[pl.ds(w * N_PAD, N_PAD)])
    plsc.subcore_barrier()

    def zsum(i, carry):
        sum_v[pl.ds(i * 16, 16)] = zero16
        return carry

    lax.fori_loop(0, RPT // 16, zsum, 0)

    for t in range(NS):
        pltpu.sync_copy(
            part.at[pl.ds((c * NS + t) * N_PAD + s * RPT, RPT)], tmp_v)

        def accum(i, carry):
            sum_v[pl.ds(i * 16, 16)] = (sum_v[pl.ds(i * 16, 16)]
                                        + tmp_v[pl.ds(i * 16, 16)])
            return carry

        lax.fori_loop(0, RPT // 16, accum, 0)

    pltpu.sync_copy(sum_v, deg_out.at[pl.ds(c * N_PAD + s * RPT, RPT)])


_sc_deg = pl.kernel(
    _deg_body,
    out_type=(
        jax.ShapeDtypeStruct((NC * NS * N_PAD,), jnp.float32),
        jax.ShapeDtypeStruct((NC * N_PAD,), jnp.float32),
    ),
    mesh=_SC_MESH,
    scratch_types=[
        pltpu.VMEM((2 * EPT,), jnp.int32),
        pltpu.VMEM((N_PAD,), jnp.float32),
        pltpu.VMEM((RPT,), jnp.float32),
        pltpu.VMEM((RPT,), jnp.float32),
    ],
    compiler_params=pltpu.CompilerParams(needs_layout_passes=False),
)


def _spmm_body(ei, z, accp, idx_v, idx2_v, rows_v, rows2_v, acc_sh, sem,
               sem2):
    c = lax.axis_index("c")
    s = lax.axis_index("s")
    pltpu.sync_copy(ei.at[pl.ds(s * EPT, EPT)], idx_v.at[pl.ds(0, EPT)])
    pltpu.sync_copy(ei.at[pl.ds(E_PAD + s * EPT, EPT)],
                    idx_v.at[pl.ds(EPT, EPT)])

    lo16 = jnp.full((16,), HN, jnp.int32) * c.astype(jnp.int32)
    hn16 = jnp.full((16,), HN, jnp.int32)
    msk16 = jnp.full((16,), TRASH - 1, jnp.int32)
    zero_i16 = jnp.zeros((16,), jnp.int32)
    zero16 = jnp.zeros((16,), jnp.float32)

    def fill(i, carry):
        for k in range(D // 16):
            rows_v[i, pl.ds(k * 16, 16)] = zero16
        return carry

    lax.fori_loop(0, CHUNK, fill, 0)

    for r in range(2):
        pltpu.sync_copy(rows_v,
                        acc_sh.at[pl.ds(s * ART + r * CHUNK, CHUNK)])
    pltpu.sync_copy(rows_v.at[pl.ds(0, ART - 2 * CHUNK)],
                    acc_sh.at[pl.ds(s * ART + 2 * CHUNK, ART - 2 * CHUNK)])
    plsc.subcore_barrier()

    def localize(base, row):
        for k in range(CHUNK // 16):
            v = idx_v[pl.ds(base + k * 16, 16)]
            lv = v - lo16
            ok = (lv >= zero_i16) & (lv < hn16)
            tr = hn16 + (v & msk16)
            idx2_v[row, pl.ds(k * 16, 16)] = jnp.where(ok, lv, tr)

    pltpu.async_copy(z.at[idx_v.at[pl.ds(0, CHUNK)]], rows_v, sem)

    def body(j, carry):
        pltpu.async_copy(z.at[idx_v.at[pl.ds(EPT + j * CHUNK, CHUNK)]],
                         rows2_v, sem2)
        localize(EPT + j * CHUNK, 0)
        pltpu.make_async_copy(z.at[idx_v.at[pl.ds(0, CHUNK)]], rows_v,
                              sem).wait()
        pltpu.sync_copy(rows_v, acc_sh.at[idx2_v.at[0]], add=True)

        @pl.when(j < NCHUNKS - 1)
        def _():
            pltpu.async_copy(
                z.at[idx_v.at[pl.ds((j + 1) * CHUNK, CHUNK)]], rows_v, sem)

        localize(j * CHUNK, 1)
        pltpu.make_async_copy(z.at[idx_v.at[pl.ds(0, CHUNK)]], rows2_v,
                              sem2).wait()
        pltpu.sync_copy(rows2_v, acc_sh.at[idx2_v.at[1]], add=True)
        return carry

    lax.fori_loop(0, NCHUNKS, body, 0)
    plsc.subcore_barrier()
    pltpu.sync_copy(acc_sh.at[pl.ds(s * OPT, OPT)],
                    accp.at[pl.ds(c * HN + s * OPT, OPT)])


_sc_spmm = pl.kernel(
    _spmm_body,
    out_type=jax.ShapeDtypeStruct((N_PAD, D), jnp.float32),
    mesh=_SC_MESH,
    scratch_types=[
        pltpu.VMEM((2 * EPT,), jnp.int32),
        pltpu.VMEM((2, CHUNK), jnp.int32),
        pltpu.VMEM((CHUNK, D), jnp.float32),
        pltpu.VMEM((CHUNK, D), jnp.float32),
        pltpu.VMEM_SHARED((ACC_ROWS, D), jnp.float32),
        pltpu.SemaphoreType.DMA,
        pltpu.SemaphoreType.DMA,
    ],
)


def _prep_body(x_ref, d0_ref, xn_ref, z0_ref, dinv_ref):
    x = x_ref[...]
    nrm = jnp.maximum(jnp.sqrt(jnp.sum(x * x, axis=1, keepdims=True)), 1e-12)
    xn = x / nrm
    deg = d0_ref[...] + 1.0
    dinv = lax.rsqrt(deg)
    xn_ref[...] = xn
    z0_ref[...] = xn * dinv
    dinv_ref[...] = dinv


_row = pl.BlockSpec((BLK, D), lambda i: (i, 0))
_col = pl.BlockSpec((BLK, 1), lambda i: (i, 0))
_wmat = pl.BlockSpec((D, D), lambda i: (0, 0))
_brow = pl.BlockSpec((1, D), lambda i: (0, 0))

_tc_prep = pl.pallas_call(
    _prep_body,
    grid=(N_PAD // BLK,),
    in_specs=[_row, _col],
    out_specs=[_row, _row, _col],
    out_shape=[
        jax.ShapeDtypeStruct((N_PAD, D), jnp.float32),
        jax.ShapeDtypeStruct((N_PAD, D), jnp.float32),
        jax.ShapeDtypeStruct((N_PAD, 1), jnp.float32),
    ],
)


def _layer_body(acc, z, dinv, s_ref, wa, ba, wb, bb, zn_ref, sn_ref):
    dv = dinv[...]
    y = (acc[...] + z[...]) * dv
    ha = jnp.dot(y, wa[...], preferred_element_type=jnp.float32) + ba[...]
    xa = jnp.where(ha >= 0, ha, 0.01 * ha)
    xb = jnp.dot(y, wb[...], preferred_element_type=jnp.float32) + bb[...]
    zn_ref[...] = xa * dv
    sn_ref[...] = s_ref[...] + xa + xb


_tc_layer = pl.pallas_call(
    _layer_body,
    grid=(N_PAD // BLK,),
    in_specs=[_row, _row, _col, _row, _wmat, _brow, _wmat, _brow],
    out_specs=[_row, _row],
    out_shape=[
        jax.ShapeDtypeStruct((N_PAD, D), jnp.float32),
        jax.ShapeDtypeStruct((N_PAD, D), jnp.float32),
    ],
)


def kernel(edge_index, id_embedding, W1, b1, W2, b2, W3, b3):
    pad = jnp.full((2, E_PAD - E), N, dtype=jnp.int32)
    ei = jnp.concatenate([edge_index, pad], axis=1).reshape(2 * E_PAD)
    x_pad = jnp.pad(id_embedding, ((0, N_PAD - N), (0, 0)))

    _, deg_flat = _sc_deg(ei)
    d0 = deg_flat[:N_PAD].reshape(N_PAD, 1)
    xn, z0, dinv = _tc_prep(x_pad, d0)

    was = jnp.stack([W1, W2])
    bas = jnp.stack([b1.reshape(1, D), b2.reshape(1, D)])
    wbs = jnp.stack([jnp.zeros_like(W3), W3])
    bbs = jnp.stack([jnp.zeros((1, D), jnp.float32), b3.reshape(1, D)])

    def step(carry, xs):
        z, s = carry
        wa, ba, wb, bb = xs
        acc = _sc_spmm(ei, z)
        zn, sn = _tc_layer(acc, z, dinv, s, wa, ba, wb, bb)
        return (zn, sn), None

    c1, _ = step((z0, xn), (was[0], bas[0], wbs[0], bbs[0]))
    (_, s_fin), _ = step(c1, (was[1], bas[1], wbs[1], bbs[1]))
    return s_fin[:N]
